# baseline (speedup 1.0000x reference)
import functools
import jax
import jax.numpy as jnp
from jax.experimental import pallas as pl

_K = 32
_ALPHA = 0.1


def _mse_kernel(x_ref, y_ref, o_ref):
    d = x_ref[...] - y_ref[...]
    o_ref[...] = jnp.sum(d * d).reshape(1, 1)


def _topk_kernel(a_ref, bot_ref, *, n_iter, n_split):
    a = a_ref[...]
    r = a.shape[0]
    s = a.shape[1]
    ai = jax.lax.bitcast_convert_type(a, jnp.int32)

    rowmax = None
    big_m = None
    for c in range(s // 128):
        x = a[:, c * 128:(c + 1) * 128]
        m1 = jnp.max(x, axis=1, keepdims=True)
        m2 = jnp.max(jnp.where(x == m1, -1.0, x), axis=1, keepdims=True)
        m2 = jnp.maximum(m2, 0.0)
        rowmax = m1 if rowmax is None else jnp.maximum(rowmax, m1)
        big_m = m2 if big_m is None else jnp.minimum(big_m, m2)

    lo_all = jax.lax.bitcast_convert_type(big_m, jnp.int32) - 1
    hi_all = jax.lax.bitcast_convert_type(rowmax, jnp.int32)

    rs = r // n_split
    parts = [ai[i * rs:(i + 1) * rs] for i in range(n_split)]
    carry0 = tuple(
        (lo_all[i * rs:(i + 1) * rs], hi_all[i * rs:(i + 1) * rs])
        for i in range(n_split)
    )

    kf = jnp.float32(_K)
    n_elem = jnp.float32(s)
    state0 = (
        jnp.int32(0),
        tuple(
            (lo, hi, jnp.full((rs, 1), n_elem), jnp.zeros((rs, 1), jnp.float32))
            for (lo, hi) in carry0
        ),
    )

    def cond(state):
        _, carry = state
        done = jnp.array(True)
        for lo, hi, _, _ in carry:
            done = jnp.logical_and(done, jnp.all(hi - lo <= 1))
        return jnp.logical_not(done)

    def body(state):
        i, carry = state
        use_interp = (i % 2) == 0
        out = []
        for (lo, hi, clo, chi), part in zip(carry, parts):
            lo_v = jax.lax.bitcast_convert_type(jnp.maximum(lo, 0), jnp.float32)
            hi_v = jax.lax.bitcast_convert_type(hi, jnp.float32)
            frac = (kf - chi) / (clo - chi)
            mid_i = jax.lax.bitcast_convert_type(hi_v - (hi_v - lo_v) * frac,
                                                 jnp.int32)
            mid_b = (lo + hi) >> 1
            mid = jnp.clip(jnp.where(use_interp, mid_i, mid_b), lo + 1, hi - 1)
            c = jnp.sum((part > mid).astype(jnp.float32), axis=1, keepdims=True)
            take = c >= _K
            lo = jnp.where(take, mid, lo)
            clo = jnp.where(take, c, clo)
            hi = jnp.where(take, hi, mid)
            chi = jnp.where(take, chi, c)
            out.append((lo, hi, clo, chi))
        return (i + 1, tuple(out))

    _, carry = jax.lax.while_loop(cond, body, state0)
    hi = jnp.concatenate([h for (_, h, _, _) in carry], axis=0)

    t = jax.lax.bitcast_convert_type(hi, jnp.float32)
    m = ai > hi
    sq = a * a
    c_gt = jnp.sum(m.astype(jnp.float32), axis=1, keepdims=True)
    s_le = jnp.sum(jnp.where(m, 0.0, sq), axis=1, keepdims=True)
    bot = s_le - (_K - c_gt) * (t * t)
    bot_ref[...] = jnp.sum(bot).reshape(1, 1, 1)


def kernel(x, y, attn):
    s = attn.shape[-1]
    rows = attn.size // s
    a2 = attn.reshape(rows, s)

    block_r = min(512, rows)
    grid = rows // block_r

    bot = pl.pallas_call(
        functools.partial(_topk_kernel, n_iter=31, n_split=2),
        grid=(grid,),
        in_specs=[pl.BlockSpec((block_r, s), lambda i: (i, 0))],
        out_specs=pl.BlockSpec((1, 1, 1), lambda i: (i, 0, 0)),
        out_shape=jax.ShapeDtypeStruct((grid, 1, 1), jnp.float32),
    )(a2)

    x2 = x.reshape(-1, x.shape[-1])
    y2 = y.reshape(-1, y.shape[-1])
    sse = pl.pallas_call(
        _mse_kernel,
        out_specs=pl.BlockSpec((1, 1), lambda: (0, 0)),
        out_shape=jax.ShapeDtypeStruct((1, 1), jnp.float32),
    )(x2, y2)

    rec_loss = sse[0, 0] / x.size
    attn_loss = jnp.sum(bot) / attn.size
    return rec_loss + _ALPHA * attn_loss


# speedup vs baseline: 2.2560x; 2.2560x over previous
import functools
import jax
import jax.numpy as jnp
from jax.experimental import pallas as pl

_K = 32
_ALPHA = 0.1


def _mse_kernel(x_ref, y_ref, o_ref):
    d = x_ref[...] - y_ref[...]
    o_ref[...] = jnp.sum(d * d).reshape(1, 1)


def _topk_kernel(a_ref, bot_ref, *, n_iter, n_split):
    a = a_ref[...]
    r = a.shape[0]
    s = a.shape[1]
    ai = jax.lax.bitcast_convert_type(a, jnp.int32)

    rowmax = None
    big_m = None
    for c in range(s // 128):
        x = a[:, c * 128:(c + 1) * 128]
        m1 = jnp.max(x, axis=1, keepdims=True)
        m2 = jnp.max(jnp.where(x == m1, -1.0, x), axis=1, keepdims=True)
        m2 = jnp.maximum(m2, 0.0)
        rowmax = m1 if rowmax is None else jnp.maximum(rowmax, m1)
        big_m = m2 if big_m is None else jnp.minimum(big_m, m2)

    lo_all = jax.lax.bitcast_convert_type(big_m, jnp.int32) - 1
    hi_all = jax.lax.bitcast_convert_type(rowmax, jnp.int32)

    rs = r // n_split
    parts = [ai[i * rs:(i + 1) * rs] for i in range(n_split)]
    carry0 = tuple(
        (lo_all[i * rs:(i + 1) * rs], hi_all[i * rs:(i + 1) * rs])
        for i in range(n_split)
    )

    kf = jnp.float32(_K)
    n_elem = jnp.float32(s)
    state0 = (
        jnp.int32(0),
        tuple(
            (lo, hi, jnp.full((rs, 1), n_elem), jnp.zeros((rs, 1), jnp.float32))
            for (lo, hi) in carry0
        ),
    )

    kf1 = jnp.float32(_K - 1)

    def cond(state):
        _, carry = state
        done = jnp.array(True)
        for lo, hi, clo, chi in carry:
            row_done = jnp.logical_or(
                jnp.logical_or(clo == kf, chi == kf1), hi - lo <= 1
            )
            done = jnp.logical_and(done, jnp.all(row_done))
        return jnp.logical_not(done)

    def body(state):
        i, carry = state
        use_interp = (i % 2) == 0
        out = []
        for (lo, hi, clo, chi), part in zip(carry, parts):
            lo_v = jax.lax.bitcast_convert_type(jnp.maximum(lo, 0), jnp.float32)
            hi_v = jax.lax.bitcast_convert_type(hi, jnp.float32)
            frac = (kf - chi) / (clo - chi)
            mid_i = jax.lax.bitcast_convert_type(hi_v - (hi_v - lo_v) * frac,
                                                 jnp.int32)
            mid_b = (lo + hi) >> 1
            mid = jnp.clip(jnp.where(use_interp, mid_i, mid_b), lo + 1, hi - 1)
            c = jnp.sum((part > mid).astype(jnp.float32), axis=1, keepdims=True)
            take = c >= _K
            lo = jnp.where(take, mid, lo)
            clo = jnp.where(take, c, clo)
            hi = jnp.where(take, hi, mid)
            chi = jnp.where(take, chi, c)
            out.append((lo, hi, clo, chi))
        return (i + 1, tuple(out))

    _, carry = jax.lax.while_loop(cond, body, state0)
    lo = jnp.concatenate([l for (l, _, _, _) in carry], axis=0)
    hi = jnp.concatenate([h for (_, h, _, _) in carry], axis=0)
    clo = jnp.concatenate([c for (_, _, c, _) in carry], axis=0)
    chi = jnp.concatenate([c for (_, _, _, c) in carry], axis=0)

    t_min = jnp.min(jnp.where(ai > lo, a, 2.0), axis=1, keepdims=True)
    t_max = jnp.max(jnp.where(ai <= hi, a, -1.0), axis=1, keepdims=True)
    t_pat = jax.lax.bitcast_convert_type(hi, jnp.float32)
    t = jnp.where(clo == kf, t_min, jnp.where(chi == kf1, t_max, t_pat))

    m = a > t
    sq = a * a
    c_gt = jnp.sum(m.astype(jnp.float32), axis=1, keepdims=True)
    s_le = jnp.sum(jnp.where(m, 0.0, sq), axis=1, keepdims=True)
    bot = s_le - (_K - c_gt) * (t * t)
    bot_ref[...] = jnp.sum(bot).reshape(1, 1, 1)


def kernel(x, y, attn):
    s = attn.shape[-1]
    rows = attn.size // s
    a2 = attn.reshape(rows, s)

    block_r = min(512, rows)
    grid = rows // block_r

    bot = pl.pallas_call(
        functools.partial(_topk_kernel, n_iter=31, n_split=2),
        grid=(grid,),
        in_specs=[pl.BlockSpec((block_r, s), lambda i: (i, 0))],
        out_specs=pl.BlockSpec((1, 1, 1), lambda i: (i, 0, 0)),
        out_shape=jax.ShapeDtypeStruct((grid, 1, 1), jnp.float32),
    )(a2)

    x2 = x.reshape(-1, x.shape[-1])
    y2 = y.reshape(-1, y.shape[-1])
    sse = pl.pallas_call(
        _mse_kernel,
        out_specs=pl.BlockSpec((1, 1), lambda: (0, 0)),
        out_shape=jax.ShapeDtypeStruct((1, 1), jnp.float32),
    )(x2, y2)

    rec_loss = sse[0, 0] / x.size
    attn_loss = jnp.sum(bot) / attn.size
    return rec_loss + _ALPHA * attn_loss
